# baseline (speedup 1.0000x reference)
import jax
import jax.numpy as jnp
from jax.experimental import pallas as pl
from jax.experimental.pallas import tpu as pltpu


def _copy_body(x_ref, e_ref, xo_ref, eo_ref, sem_x, sem_e):
    cx = pltpu.make_async_copy(x_ref, xo_ref, sem_x)
    ce = pltpu.make_async_copy(e_ref, eo_ref, sem_e)
    cx.start()
    ce.start()
    cx.wait()
    ce.wait()


def kernel(x, edge_index, edge_attr):
    del edge_index
    return tuple(
        pl.pallas_call(
            _copy_body,
            in_specs=[
                pl.BlockSpec(memory_space=pl.ANY),
                pl.BlockSpec(memory_space=pl.ANY),
            ],
            out_specs=[
                pl.BlockSpec(memory_space=pl.ANY),
                pl.BlockSpec(memory_space=pl.ANY),
            ],
            out_shape=[
                jax.ShapeDtypeStruct(x.shape, x.dtype),
                jax.ShapeDtypeStruct(edge_attr.shape, edge_attr.dtype),
            ],
            scratch_shapes=[pltpu.SemaphoreType.DMA, pltpu.SemaphoreType.DMA],
        )(x, edge_attr)
    )


# speedup vs baseline: 16.9539x; 16.9539x over previous
import jax
import jax.numpy as jnp
from jax.experimental import pallas as pl
from jax.experimental.pallas import tpu as pltpu

_GRID = 25


def _copy_body(xb, eb, xob, eob):
    xob[...] = xb[...]
    eob[...] = eb[...]


def kernel(x, edge_index, edge_attr):
    del edge_index
    e_shape = edge_attr.shape
    ew = edge_attr.reshape(e_shape[0] * e_shape[1] // 128, 128)
    xb = x.shape[0] // _GRID
    eb = ew.shape[0] // _GRID
    xo, eo = pl.pallas_call(
        _copy_body,
        grid=(_GRID,),
        in_specs=[
            pl.BlockSpec((xb, 128), lambda i: (i, 0)),
            pl.BlockSpec((eb, 128), lambda i: (i, 0)),
        ],
        out_specs=[
            pl.BlockSpec((xb, 128), lambda i: (i, 0)),
            pl.BlockSpec((eb, 128), lambda i: (i, 0)),
        ],
        out_shape=[
            jax.ShapeDtypeStruct(x.shape, x.dtype),
            jax.ShapeDtypeStruct(ew.shape, ew.dtype),
        ],
    )(x, ew)
    return (xo, eo.reshape(e_shape))


# speedup vs baseline: 17.4649x; 1.0301x over previous
import functools

import jax
import jax.numpy as jnp
from jax.experimental import pallas as pl
from jax.experimental.pallas import tpu as pltpu
from jax.experimental.pallas import tpu_sc as plsc

_NC, _NS = 2, 16
_NW = _NC * _NS
_CHUNKS = 25


def _sc_copy_body(e_hbm, eo_hbm, buf0, buf1, sin0, sin1, sout0, sout1):
    wid = jax.lax.axis_index("s") * _NC + jax.lax.axis_index("c")
    rows = e_hbm.shape[0]
    per_w = rows // _NW
    ch = per_w // _CHUNKS
    base = wid * per_w
    bufs = (buf0, buf1)
    sins = (sin0, sin1)
    souts = (sout0, sout1)

    def dma_in(i):
        return pltpu.make_async_copy(
            e_hbm.at[pl.ds(base + i * ch, ch)], bufs[i % 2], sins[i % 2]
        )

    def dma_out(i):
        return pltpu.make_async_copy(
            bufs[i % 2], eo_hbm.at[pl.ds(base + i * ch, ch)], souts[i % 2]
        )

    dma_in(0).start()
    for i in range(_CHUNKS):
        if i + 1 < _CHUNKS:
            if i >= 1:
                dma_out(i - 1).wait()
            dma_in(i + 1).start()
        dma_in(i).wait()
        dma_out(i).start()
    if _CHUNKS >= 2:
        dma_out(_CHUNKS - 2).wait()
    dma_out(_CHUNKS - 1).wait()


def _tc_copy_body(xb, xob):
    xob[...] = xb[...]


def kernel(x, edge_index, edge_attr):
    del edge_index

    rows = edge_attr.shape[0]
    ch = rows // _NW // _CHUNKS
    sc_copy = pl.kernel(
        _sc_copy_body,
        out_type=jax.ShapeDtypeStruct(edge_attr.shape, edge_attr.dtype),
        mesh=plsc.VectorSubcoreMesh(core_axis_name="c", subcore_axis_name="s"),
        scratch_types=[
            pltpu.VMEM((ch, edge_attr.shape[1]), edge_attr.dtype),
            pltpu.VMEM((ch, edge_attr.shape[1]), edge_attr.dtype),
            pltpu.SemaphoreType.DMA,
            pltpu.SemaphoreType.DMA,
            pltpu.SemaphoreType.DMA,
            pltpu.SemaphoreType.DMA,
        ],
    )
    eo = sc_copy(edge_attr)

    grid = 10
    xb = x.shape[0] // grid
    xo = pl.pallas_call(
        _tc_copy_body,
        grid=(grid,),
        in_specs=[pl.BlockSpec((xb, x.shape[1]), lambda i: (i, 0))],
        out_specs=pl.BlockSpec((xb, x.shape[1]), lambda i: (i, 0)),
        out_shape=jax.ShapeDtypeStruct(x.shape, x.dtype),
    )(x)
    return (xo, eo)
